# baseline (speedup 1.0000x reference)
import functools

import jax
import jax.numpy as jnp
from jax import lax
from jax.experimental import pallas as pl
from jax.experimental.pallas import tpu as pltpu
from jax.experimental.pallas import tpu_sc as plsc

_BLK = 1024
_CHUNKS = 1


def _dist_argmax_body(x_ref, e_ref, dist_ref, ind_ref):
    xb = x_ref[...]
    eb = e_ref[...]
    x2 = jnp.sum(xb * xb, axis=1, keepdims=True)
    e2 = jnp.sum(eb * eb, axis=1)
    xe2 = lax.dot_general(xb + xb, eb, (((1,), (1,)), ((), ())),
                          preferred_element_type=jnp.float32)
    dist = (xe2 - x2) - e2
    dist_ref[...] = dist
    k = dist.shape[1]
    m = jnp.max(dist, axis=1, keepdims=True)
    iota = lax.broadcasted_iota(jnp.int32, dist.shape, 1).astype(jnp.float32)
    ind = jnp.min(jnp.where(dist == m, iota, float(k)), axis=1)
    ind_ref[...] = ind.astype(jnp.int32).reshape(ind_ref.shape)


def _dist_argmax_chunk(x_flat, embed2d, chunk, n_chunks, bufs=None):
    bn, d = x_flat.shape
    k = embed2d.shape[0]
    steps = bn // n_chunks // _BLK
    base = chunk * steps

    in_specs = [
        pl.BlockSpec((_BLK, d), lambda i, b=base: (i + b, 0)),
        pl.BlockSpec((k, d), lambda i: (0, 0)),
    ]
    operands = [x_flat, embed2d]
    aliases = {}
    if bufs is not None:
        in_specs += [pl.BlockSpec(memory_space=pl.ANY),
                     pl.BlockSpec(memory_space=pl.ANY)]
        operands += list(bufs)
        aliases = {2: 0, 3: 1}

    def body(x_ref, e_ref, *refs):
        dist_ref, ind_ref = refs[-2], refs[-1]
        _dist_argmax_body(x_ref, e_ref, dist_ref, ind_ref)

    return pl.pallas_call(
        body,
        grid=(steps,),
        in_specs=in_specs,
        out_specs=[
            pl.BlockSpec((_BLK, k), lambda i, b=base: (i + b, 0)),
            pl.BlockSpec((_BLK // 128, 128), lambda i, b=base: (i + b, 0)),
        ],
        out_shape=[
            jax.ShapeDtypeStruct((bn, k), jnp.float32),
            jax.ShapeDtypeStruct((bn // 128, 128), jnp.int32),
        ],
        input_output_aliases=aliases,
    )(*operands)


def _sc_gather(table, idx):
    info = plsc.get_sparse_core_info()
    nc, ns = info.num_cores, info.num_subcores
    nw = nc * ns
    bn = idx.shape[0]
    k, dp = table.shape
    d = 64
    b_per_w = bn // nw
    nbatch = 4
    b_sub = b_per_w // nbatch
    mesh = plsc.VectorSubcoreMesh(core_axis_name="c", subcore_axis_name="s")

    @functools.partial(
        pl.kernel, mesh=mesh,
        out_type=jax.ShapeDtypeStruct((bn, dp), jnp.float32),
        scratch_types=[
            pltpu.VMEM((b_per_w,), jnp.int32),
            pltpu.VMEM((b_sub, dp), jnp.float32),
            pltpu.VMEM((b_sub, dp), jnp.float32),
            pltpu.SemaphoreType.DMA,
            pltpu.SemaphoreType.DMA,
            pltpu.SemaphoreType.DMA,
            pltpu.SemaphoreType.DMA,
        ],
    )
    def gk(table_hbm, idx_hbm, out_hbm, idx_v, r0, r1, g0, g1, w0, w1):
        wid = lax.axis_index("s") * nc + lax.axis_index("c")
        base = wid * b_per_w
        pltpu.sync_copy(idx_hbm.at[pl.ds(base, b_per_w)], idx_v)

        rows = [r0, r1]
        gsem = [g0, g1]
        wsem = [w0, w1]
        gh = pltpu.async_copy(
            table_hbm.at[idx_v.at[pl.ds(0, b_sub)]], rows[0], gsem[0])
        wh = [None, None]
        for bi in range(nbatch):
            cur = bi % 2
            nxt = 1 - cur
            gh_cur = gh
            if bi + 1 < nbatch:
                if wh[nxt] is not None:
                    wh[nxt].wait()
                gh = pltpu.async_copy(
                    table_hbm.at[idx_v.at[pl.ds((bi + 1) * b_sub, b_sub)]],
                    rows[nxt], gsem[nxt])
            gh_cur.wait()
            wh[cur] = pltpu.async_copy(
                rows[cur], out_hbm.at[pl.ds(base + bi * b_sub, b_sub)],
                wsem[cur])
        wh[(nbatch - 2) % 2].wait()
        wh[(nbatch - 1) % 2].wait()

    return gk(table, idx)

    return gk(table, idx)


def kernel(x, embed):
    b, n, d = x.shape
    h, k, _ = embed.shape
    bn = b * n
    x_flat = x.reshape(bn, d).astype(jnp.float32)
    embed2d = embed.reshape(k, d)
    table_pad = jnp.pad(embed2d, ((0, 0), (0, 128 - d)))

    rows_per_chunk = bn // _CHUNKS
    bufs = None
    q_parts = []
    for c in range(_CHUNKS):
        bufs = _dist_argmax_chunk(x_flat, embed2d, c, _CHUNKS, bufs)
        ind_chunk = lax.dynamic_slice(
            bufs[1].reshape(bn), (c * rows_per_chunk,), (rows_per_chunk,))
        q_parts.append(_sc_gather(table_pad, ind_chunk))

    dist_buf, ind_buf = bufs
    q128 = jnp.concatenate(q_parts, axis=0) if _CHUNKS > 1 else q_parts[0]
    quantize = q128[:, :d]
    return (quantize.reshape(b, n, d),
            ind_buf.reshape(b, n),
            dist_buf.reshape(h, bn, k))


# speedup vs baseline: 1.0056x; 1.0056x over previous
import functools

import jax
import jax.numpy as jnp
from jax import lax
from jax.experimental import pallas as pl
from jax.experimental.pallas import tpu as pltpu
from jax.experimental.pallas import tpu_sc as plsc

_BLK = 1024
_CHUNKS = 1


def _dist_argmax_body(x_ref, e_ref, dist_ref, ind_ref):
    xb = x_ref[...]
    eb = e_ref[...]
    x2 = jnp.sum(xb * xb, axis=1, keepdims=True)
    e2 = jnp.sum(eb * eb, axis=1)
    xe2 = lax.dot_general(xb + xb, eb, (((1,), (1,)), ((), ())),
                          preferred_element_type=jnp.float32)
    dist = (xe2 - x2) - e2
    dist_ref[...] = dist
    k = dist.shape[1]
    m = jnp.max(dist, axis=1, keepdims=True)
    iota = lax.broadcasted_iota(jnp.int32, dist.shape, 1).astype(jnp.float32)
    ind = jnp.min(jnp.where(dist == m, iota, float(k)), axis=1)
    ind_ref[...] = ind.astype(jnp.int32).reshape(ind_ref.shape)


def _dist_argmax_chunk(x_flat, embed2d, chunk, n_chunks, bufs=None):
    bn, d = x_flat.shape
    k = embed2d.shape[0]
    steps = bn // n_chunks // _BLK
    base = chunk * steps

    in_specs = [
        pl.BlockSpec((_BLK, d), lambda i, b=base: (i + b, 0)),
        pl.BlockSpec((k, d), lambda i: (0, 0)),
    ]
    operands = [x_flat, embed2d]
    aliases = {}
    if bufs is not None:
        in_specs += [pl.BlockSpec(memory_space=pl.ANY),
                     pl.BlockSpec(memory_space=pl.ANY)]
        operands += list(bufs)
        aliases = {2: 0, 3: 1}

    def body(x_ref, e_ref, *refs):
        dist_ref, ind_ref = refs[-2], refs[-1]
        _dist_argmax_body(x_ref, e_ref, dist_ref, ind_ref)

    return pl.pallas_call(
        body,
        grid=(steps,),
        in_specs=in_specs,
        out_specs=[
            pl.BlockSpec((_BLK, k), lambda i, b=base: (i + b, 0)),
            pl.BlockSpec((_BLK // 128, 128), lambda i, b=base: (i + b, 0)),
        ],
        out_shape=[
            jax.ShapeDtypeStruct((bn, k), jnp.float32),
            jax.ShapeDtypeStruct((bn // 128, 128), jnp.int32),
        ],
        input_output_aliases=aliases,
    )(*operands)


def _sc_gather(table, idx):
    info = plsc.get_sparse_core_info()
    nc, ns = info.num_cores, info.num_subcores
    nw = nc * ns
    bn = idx.shape[0]
    k, dp = table.shape
    b_per_w = bn // nw
    nbatch = 2
    b_sub = b_per_w // nbatch
    mesh = plsc.VectorSubcoreMesh(core_axis_name="c", subcore_axis_name="s")

    @functools.partial(
        pl.kernel, mesh=mesh,
        out_type=jax.ShapeDtypeStruct((bn, dp), jnp.float32),
        scratch_types=[
            pltpu.VMEM((b_per_w,), jnp.int32),
            pltpu.VMEM((b_sub, dp), jnp.float32),
            pltpu.SemaphoreType.DMA,
        ],
        compiler_params=pltpu.CompilerParams(use_tc_tiling_on_sc=True),
    )
    def gk(table_hbm, idx_hbm, out_hbm, idx_v, rows_v, sem):
        wid = lax.axis_index("s") * nc + lax.axis_index("c")
        base = wid * b_per_w
        pltpu.sync_copy(idx_hbm.at[pl.ds(base, b_per_w)], idx_v)

        def batch(bi, _):
            pltpu.async_copy(
                table_hbm.at[idx_v.at[pl.ds(bi * b_sub, b_sub)]],
                rows_v, sem).wait()
            pltpu.sync_copy(
                rows_v, out_hbm.at[pl.ds(base + bi * b_sub, b_sub)])
            return ()

        lax.fori_loop(0, nbatch, batch, ())

    return gk(table, idx)

    return gk(table, idx)


def kernel(x, embed):
    b, n, d = x.shape
    h, k, _ = embed.shape
    bn = b * n
    x_flat = x.reshape(bn, d).astype(jnp.float32)
    embed2d = embed.reshape(k, d)
    table_pad = jnp.pad(embed2d, ((0, 0), (0, 128 - d)))

    rows_per_chunk = bn // _CHUNKS
    bufs = None
    q_parts = []
    for c in range(_CHUNKS):
        bufs = _dist_argmax_chunk(x_flat, embed2d, c, _CHUNKS, bufs)
        ind_chunk = lax.dynamic_slice(
            bufs[1].reshape(bn), (c * rows_per_chunk,), (rows_per_chunk,))
        q_parts.append(_sc_gather(table_pad, ind_chunk))

    dist_buf, ind_buf = bufs
    q128 = jnp.concatenate(q_parts, axis=0) if _CHUNKS > 1 else q_parts[0]
    quantize = q128[:, :d]
    return (quantize.reshape(b, n, d),
            ind_buf.reshape(b, n),
            dist_buf.reshape(h, bn, k))


# speedup vs baseline: 1.1033x; 1.0972x over previous
import functools

import jax
import jax.numpy as jnp
from jax import lax
from jax.experimental import pallas as pl
from jax.experimental.pallas import tpu as pltpu
from jax.experimental.pallas import tpu_sc as plsc

_BLK = 2048
_CHUNKS = 1


def _dist_argmax_body(x_ref, e_ref, dist_ref, ind_ref):
    xb = x_ref[...]
    eb = e_ref[...]
    x2 = jnp.sum(xb * xb, axis=1, keepdims=True)
    e2 = jnp.sum(eb * eb, axis=1)
    xe2 = lax.dot_general(xb + xb, eb, (((1,), (1,)), ((), ())),
                          preferred_element_type=jnp.float32)
    dist = (xe2 - x2) - e2
    dist_ref[...] = dist
    k = dist.shape[1]
    m = jnp.max(dist, axis=1, keepdims=True)
    iota = lax.broadcasted_iota(jnp.int32, dist.shape, 1).astype(jnp.float32)
    ind = jnp.min(jnp.where(dist == m, iota, float(k)), axis=1)
    ind_ref[...] = ind.astype(jnp.int32).reshape(ind_ref.shape)


def _dist_argmax_chunk(x_flat, embed2d, chunk, n_chunks, bufs=None):
    bn, d = x_flat.shape
    k = embed2d.shape[0]
    steps = bn // n_chunks // _BLK
    base = chunk * steps

    in_specs = [
        pl.BlockSpec((_BLK, d), lambda i, b=base: (i + b, 0)),
        pl.BlockSpec((k, d), lambda i: (0, 0)),
    ]
    operands = [x_flat, embed2d]
    aliases = {}
    if bufs is not None:
        in_specs += [pl.BlockSpec(memory_space=pl.ANY),
                     pl.BlockSpec(memory_space=pl.ANY)]
        operands += list(bufs)
        aliases = {2: 0, 3: 1}

    def body(x_ref, e_ref, *refs):
        dist_ref, ind_ref = refs[-2], refs[-1]
        _dist_argmax_body(x_ref, e_ref, dist_ref, ind_ref)

    return pl.pallas_call(
        body,
        grid=(steps,),
        in_specs=in_specs,
        out_specs=[
            pl.BlockSpec((_BLK, k), lambda i, b=base: (i + b, 0)),
            pl.BlockSpec((_BLK // 128, 128), lambda i, b=base: (i + b, 0)),
        ],
        out_shape=[
            jax.ShapeDtypeStruct((bn, k), jnp.float32),
            jax.ShapeDtypeStruct((bn // 128, 128), jnp.int32),
        ],
        input_output_aliases=aliases,
    )(*operands)


def _sc_gather(table, idx):
    info = plsc.get_sparse_core_info()
    nc, ns = info.num_cores, info.num_subcores
    nw = nc * ns
    bn = idx.shape[0]
    k, dp = table.shape
    b_per_w = bn // nw
    nbatch = 2
    b_sub = b_per_w // nbatch
    mesh = plsc.VectorSubcoreMesh(core_axis_name="c", subcore_axis_name="s")

    @functools.partial(
        pl.kernel, mesh=mesh,
        out_type=jax.ShapeDtypeStruct((bn, dp), jnp.float32),
        scratch_types=[
            pltpu.VMEM((b_per_w,), jnp.int32),
            pltpu.VMEM((b_sub, dp), jnp.float32),
            pltpu.SemaphoreType.DMA,
        ],
        compiler_params=pltpu.CompilerParams(use_tc_tiling_on_sc=True),
    )
    def gk(table_hbm, idx_hbm, out_hbm, idx_v, rows_v, sem):
        wid = lax.axis_index("s") * nc + lax.axis_index("c")
        base = wid * b_per_w
        pltpu.sync_copy(idx_hbm.at[pl.ds(base, b_per_w)], idx_v)

        def batch(bi, _):
            pltpu.async_copy(
                table_hbm.at[idx_v.at[pl.ds(bi * b_sub, b_sub)]],
                rows_v, sem).wait()
            pltpu.sync_copy(
                rows_v, out_hbm.at[pl.ds(base + bi * b_sub, b_sub)])
            return ()

        lax.fori_loop(0, nbatch, batch, ())

    return gk(table, idx)


def kernel(x, embed):
    b, n, d = x.shape
    h, k, _ = embed.shape
    bn = b * n
    x_flat = x.reshape(bn, d).astype(jnp.float32)
    embed2d = embed.reshape(k, d)
    table_pad = jnp.pad(embed2d, ((0, 0), (0, 128 - d)))

    rows_per_chunk = bn // _CHUNKS
    bufs = None
    q_parts = []
    for c in range(_CHUNKS):
        bufs = _dist_argmax_chunk(x_flat, embed2d, c, _CHUNKS, bufs)
        ind_chunk = lax.dynamic_slice(
            bufs[1].reshape(bn), (c * rows_per_chunk,), (rows_per_chunk,))
        q_parts.append(_sc_gather(table_pad, ind_chunk))

    dist_buf, ind_buf = bufs
    q128 = jnp.concatenate(q_parts, axis=0) if _CHUNKS > 1 else q_parts[0]
    quantize = q128[:, :d]
    return (quantize.reshape(b, n, d),
            ind_buf.reshape(b, n),
            dist_buf.reshape(h, bn, k))
